# baseline (speedup 1.0000x reference)
import jax
import jax.numpy as jnp
from jax import lax
from jax.experimental import pallas as pl
from jax.experimental.pallas import tpu as pltpu
from jax.experimental.pallas import tpu_sc as plsc

B, L = 16384, 200
DAY_VOCAB, TIME_VOCAB = 7, 96
D = 64
N = B * L
NW = 32
GROUPS = B // 128
GPW = GROUPS // NW
ROWS_PER_W = N // NW
UNIT = 32
UPW = ROWS_PER_W // UNIT
UPG = 128 * L // UNIT
NBUF = 12


def _sc_body(dt_hbm, cat_hbm, out_hbm, *s):
    ibuf = s[0]
    idxf = s[1 : 1 + NBUF]
    rows = s[1 + NBUF : 1 + 2 * NBUF]
    gsem = s[1 + 2 * NBUF : 1 + 3 * NBUF]
    wsem = s[1 + 3 * NBUF : 1 + 4 * NBUF]
    cat_sp = s[1 + 4 * NBUF]
    nc = 2
    wid = lax.axis_index("s") * nc + lax.axis_index("c")
    row0_w = wid * ROWS_PER_W
    iota = lax.broadcasted_iota(jnp.int32, (16,), 0)

    @pl.when(lax.axis_index("s") == 0)
    def _():
        pltpu.sync_copy(cat_hbm, cat_sp)

    plsc.subcore_barrier()

    def load_group(nu):
        gidx = wid * GPW + nu // UPG
        pltpu.sync_copy(dt_hbm.at[:, gidx, :], ibuf)

    def compute_idxf(nu, b):
        r0 = row0_w + nu * UNIT
        for k in range((UNIT + 15) // 16):
            r = r0 + k * 16 + iota
            bat = r // L
            l = r - bat * L
            col = bat & 127
            d = plsc.load_gather(ibuf, [l, col])
            t = plsc.load_gather(ibuf, [l, col + 128])
            idxf[b][0, pl.ds(k * 16, 16)] = d * TIME_VOCAB + t

    def gcopy(b):
        return pltpu.make_async_copy(
            cat_sp.at[idxf[b].at[0, pl.ds(0, UNIT)]], rows[b], gsem[b]
        )

    def wcopy(b, u):
        return pltpu.make_async_copy(
            rows[b], out_hbm.at[pl.ds(row0_w + u * UNIT, UNIT)], wsem[b]
        )

    load_group(0)
    for j in range(NBUF):
        compute_idxf(j, j)
        gcopy(j).start()

    def step(p, carry):
        for b in range(NBUF):
            u = NBUF * p + b

            @pl.when(u < UPW)
            def _(u=u, b=b):
                gcopy(b).wait()
                wcopy(b, u).start()
            nu = u + NBUF

            @pl.when(nu < UPW)
            def _(nu=nu, b=b):
                @pl.when(nu % UPG == 0)
                def _():
                    load_group(nu)

                compute_idxf(nu, b)
                wcopy(b, nu).wait()
                gcopy(b).start()

        return carry

    lax.fori_loop(0, (UPW + NBUF - 1) // NBUF, step, None)
    for b in range(NBUF):
        wcopy(b, 0).wait()


@jax.jit
def _daytime_sc(dt3, cat):
    mesh = plsc.VectorSubcoreMesh(core_axis_name="c", subcore_axis_name="s")
    return pl.kernel(
        _sc_body,
        out_type=jax.ShapeDtypeStruct((N, 2 * D), jnp.float32),
        mesh=mesh,
        compiler_params=pltpu.CompilerParams(
            needs_layout_passes=False, use_tc_tiling_on_sc=True
        ),
        scratch_types=(
            [pltpu.VMEM((L, 256), jnp.int32)]
            + [pltpu.VMEM((1, 128), jnp.int32)] * NBUF
            + [pltpu.VMEM((UNIT, 2 * D), jnp.float32)] * NBUF
            + [pltpu.SemaphoreType.DMA] * (2 * NBUF)
            + [pltpu.VMEM_SHARED((DAY_VOCAB * TIME_VOCAB, 2 * D), jnp.float32)]
        ),
    )(dt3, cat)


def kernel(daytime, emb_day, emb_time):
    cat = jnp.concatenate(
        (
            jnp.broadcast_to(emb_day[:, None, :], (DAY_VOCAB, TIME_VOCAB, D)),
            jnp.broadcast_to(emb_time[None, :, :], (DAY_VOCAB, TIME_VOCAB, D)),
        ),
        axis=-1,
    ).reshape(DAY_VOCAB * TIME_VOCAB, 2 * D)
    dt3 = (
        daytime.reshape(B // 128, 128, L, 2)
        .transpose(2, 0, 3, 1)
        .reshape(L, B // 128, 256)
    )
    out = _daytime_sc(dt3, cat)
    return out.reshape(B, L, 2 * D)


# speedup vs baseline: 1.0328x; 1.0328x over previous
import jax
import jax.numpy as jnp
from jax import lax
from jax.experimental import pallas as pl
from jax.experimental.pallas import tpu as pltpu
from jax.experimental.pallas import tpu_sc as plsc

B, L = 16384, 200
DAY_VOCAB, TIME_VOCAB = 7, 96
D = 64
N = B * L
NW = 32
GROUPS = B // 128
GPW = GROUPS // NW
ROWS_PER_W = N // NW
UNIT = 64
UPW = ROWS_PER_W // UNIT
UPG = 128 * L // UNIT
NBUF = 8


def _sc_body(dt_hbm, cat_hbm, out_hbm, *s):
    ibuf = s[0]
    idxf = s[1 : 1 + NBUF]
    rows = s[1 + NBUF : 1 + 2 * NBUF]
    gsem = s[1 + 2 * NBUF : 1 + 3 * NBUF]
    wsem = s[1 + 3 * NBUF : 1 + 4 * NBUF]
    cat_sp = s[1 + 4 * NBUF]
    nc = 2
    wid = lax.axis_index("s") * nc + lax.axis_index("c")
    row0_w = wid * ROWS_PER_W
    iota = lax.broadcasted_iota(jnp.int32, (16,), 0)

    @pl.when(lax.axis_index("s") == 0)
    def _():
        pltpu.sync_copy(cat_hbm, cat_sp)

    plsc.subcore_barrier()

    def load_group(nu):
        gidx = wid * GPW + nu // UPG
        pltpu.sync_copy(dt_hbm.at[:, gidx, :], ibuf)

    def compute_idxf(nu, b):
        r0 = row0_w + nu * UNIT
        for k in range((UNIT + 15) // 16):
            r = r0 + k * 16 + iota
            bat = r // L
            l = r - bat * L
            col = bat & 127
            d = plsc.load_gather(ibuf, [l, col])
            t = plsc.load_gather(ibuf, [l, col + 128])
            idxf[b][0, pl.ds(k * 16, 16)] = d * TIME_VOCAB + t

    def gcopy(b):
        return pltpu.make_async_copy(
            cat_sp.at[idxf[b].at[0, pl.ds(0, UNIT)]], rows[b], gsem[b]
        )

    def wcopy(b, u):
        return pltpu.make_async_copy(
            rows[b], out_hbm.at[pl.ds(row0_w + u * UNIT, UNIT)], wsem[b]
        )

    load_group(0)
    for j in range(NBUF):
        compute_idxf(j, j)
        gcopy(j).start()

    def step(p, carry):
        for b in range(NBUF):
            u = NBUF * p + b

            @pl.when(u < UPW)
            def _(u=u, b=b):
                gcopy(b).wait()
                wcopy(b, u).start()
            nu = u + NBUF

            @pl.when(nu < UPW)
            def _(nu=nu, b=b):
                @pl.when(nu % UPG == 0)
                def _():
                    load_group(nu)

                compute_idxf(nu, b)
                wcopy(b, nu).wait()
                gcopy(b).start()

        return carry

    lax.fori_loop(0, (UPW + NBUF - 1) // NBUF, step, None)
    for b in range(NBUF):
        wcopy(b, 0).wait()


@jax.jit
def _daytime_sc(dt3, cat):
    mesh = plsc.VectorSubcoreMesh(core_axis_name="c", subcore_axis_name="s")
    return pl.kernel(
        _sc_body,
        out_type=jax.ShapeDtypeStruct((N, 2 * D), jnp.float32),
        mesh=mesh,
        compiler_params=pltpu.CompilerParams(
            needs_layout_passes=False, use_tc_tiling_on_sc=True
        ),
        scratch_types=(
            [pltpu.VMEM((L, 256), jnp.int32)]
            + [pltpu.VMEM((1, 128), jnp.int32)] * NBUF
            + [pltpu.VMEM((UNIT, 2 * D), jnp.float32)] * NBUF
            + [pltpu.SemaphoreType.DMA] * (2 * NBUF)
            + [pltpu.VMEM_SHARED((DAY_VOCAB * TIME_VOCAB, 2 * D), jnp.float32)]
        ),
    )(dt3, cat)


def kernel(daytime, emb_day, emb_time):
    cat = jnp.concatenate(
        (
            jnp.broadcast_to(emb_day[:, None, :], (DAY_VOCAB, TIME_VOCAB, D)),
            jnp.broadcast_to(emb_time[None, :, :], (DAY_VOCAB, TIME_VOCAB, D)),
        ),
        axis=-1,
    ).reshape(DAY_VOCAB * TIME_VOCAB, 2 * D)
    dt3 = (
        daytime.reshape(B // 128, 128, L, 2)
        .transpose(2, 0, 3, 1)
        .reshape(L, B // 128, 256)
    )
    out = _daytime_sc(dt3, cat)
    return out.reshape(B, L, 2 * D)
